# baseline (speedup 1.0000x reference)
import functools
import jax
import jax.numpy as jnp
from jax import lax
from jax.experimental import pallas as pl
from jax.experimental.pallas import tpu as pltpu, tpu_sc as plsc

V = 1000000
D = 16
B = 16384
NCTX = 5
NC, NS, L = 2, 16, 16
NW = NC * NS
BW = B // NW
CW = BW * NCTX
NCH = 8
CB = BW // NCH
CC = CB * NCTX
NGRP = CC // L


def _fire(ttab, ctab, tgt_row, ctx_row, we_big, ce_big, sem, c, buf):
    descs = [pltpu.make_async_copy(
        ttab.at[tgt_row.at[pl.ds(c * CB, CB)]], we_big.at[buf], sem)]
    for j, n in ((0, 128), (128, 128), (256, 64)):
        descs.append(pltpu.make_async_copy(
            ctab.at[ctx_row.at[pl.ds(c * CC + j, n)]],
            ce_big.at[buf, pl.ds(j, n)], sem))
    for d in descs:
        d.start()
    return descs


def _body(tgt_hbm, ctx_hbm, ttab, ctab, out_hbm,
          tgt_idx, ctx_idx, tgt_row, ctx_row, we_big, ce_big, out_v,
          sem0, sem1):
    wid = lax.axis_index("s") * NC + lax.axis_index("c")

    pltpu.sync_copy(tgt_hbm.at[pl.ds(wid * BW, BW)], tgt_idx)
    pltpu.sync_copy(ctx_hbm.at[pl.ds(wid * CW, CW)], ctx_idx)

    def shift_t(i, _):
        tgt_row[pl.ds(pl.multiple_of(i * L, L), L)] = (
            tgt_idx[pl.ds(pl.multiple_of(i * L, L), L)] >> 3)
        return 0
    lax.fori_loop(0, BW // L, shift_t, 0)

    def shift_c(i, _):
        ctx_row[pl.ds(pl.multiple_of(i * L, L), L)] = (
            ctx_idx[pl.ds(pl.multiple_of(i * L, L), L)] >> 3)
        return 0
    lax.fori_loop(0, CW // L, shift_c, 0)

    lane = lax.iota(jnp.int32, L)
    sems = (sem0, sem1)
    _fire(ttab, ctab, tgt_row, ctx_row, we_big, ce_big, sems[0], 0, 0)

    for c in range(NCH):
        p = c % 2
        if c + 1 < NCH:
            _fire(ttab, ctab, tgt_row, ctx_row, we_big, ce_big,
                  sems[1 - p], c + 1, 1 - p)
        for d2 in _make_waiters(ttab, ctab, tgt_row, ctx_row,
                                we_big, ce_big, sems[p], c, p):
            d2.wait()

        pvec = jnp.full((L,), p, jnp.int32)

        def group(g, _, c=c, pvec=pvec):
            o = g * L + lane
            b = o // NCTX
            idx_c = ctx_idx[pl.ds(pl.multiple_of(c * CC, 8) + g * L, L)]
            idx_t = plsc.load_gather(tgt_idx, [c * CB + b])
            colc = (idx_c & 7) << 4
            colt = (idx_t & 7) << 4
            acc = jnp.zeros((L,), jnp.float32)
            for d in range(D):
                ce = plsc.load_gather(ce_big, [pvec, o, colc + d])
                we = plsc.load_gather(we_big, [pvec, b, colt + d])
                acc = acc + ce * we
            out_v[pl.ds(pl.multiple_of(c * CC, 8) + g * L, L)] = acc
            return 0

        lax.fori_loop(0, NGRP, group, 0)

    pltpu.sync_copy(out_v, out_hbm.at[pl.ds(wid * CW, CW)])


def _make_waiters(ttab, ctab, tgt_row, ctx_row, we_big, ce_big, sem, c, buf):
    descs = [pltpu.make_async_copy(
        ttab.at[tgt_row.at[pl.ds(c * CB, CB)]], we_big.at[buf], sem)]
    for j, n in ((0, 128), (128, 128), (256, 64)):
        descs.append(pltpu.make_async_copy(
            ctab.at[ctx_row.at[pl.ds(c * CC + j, n)]],
            ce_big.at[buf, pl.ds(j, n)], sem))
    return descs


@functools.partial(jax.jit, static_argnames=())
def kernel(target, context, target_table, context_table):
    mesh = plsc.VectorSubcoreMesh(
        core_axis_name="c", subcore_axis_name="s",
        num_cores=NC, num_subcores=NS)
    run = pl.kernel(
        _body,
        out_type=jax.ShapeDtypeStruct((B * NCTX,), jnp.float32),
        mesh=mesh,
        scratch_types=[
            pltpu.VMEM((BW,), jnp.int32),
            pltpu.VMEM((CW,), jnp.int32),
            pltpu.VMEM((BW,), jnp.int32),
            pltpu.VMEM((CW,), jnp.int32),
            pltpu.VMEM((2, CB, 128), jnp.float32),
            pltpu.VMEM((2, CC, 128), jnp.float32),
            pltpu.VMEM((CW,), jnp.float32),
            pltpu.SemaphoreType.DMA,
            pltpu.SemaphoreType.DMA,
        ],
        compiler_params=pltpu.CompilerParams(needs_layout_passes=False),
    )
    out = run(target.reshape(B).astype(jnp.int32),
              context.reshape(B * NCTX).astype(jnp.int32),
              target_table.reshape(V // 8, 128),
              context_table.reshape(V // 8, 128))
    return out.reshape(B, NCTX)
